# baseline (speedup 1.0000x reference)
import functools

import jax
import jax.numpy as jnp
from jax import lax
from jax.experimental import pallas as pl
from jax.experimental.pallas import tpu as pltpu
from jax.experimental.pallas import tpu_sc as plsc

S = 2048
D = 1024
E = 64
CHUNK = 128
P2 = S + 8 * E
XROWS = P2 + CHUNK
RB = 256


def _rank_kernel(u_col_ref, u_row_ref, sp_ref, g_ref):
    u_r = u_row_ref[...]
    u_c = u_col_ref[...]
    er = lax.broadcasted_iota(jnp.int32, (1, E), 1)
    ec = lax.broadcasted_iota(jnp.int32, (E, 1), 0)

    hist = jnp.sum((u_c == er).astype(jnp.int32), axis=0, keepdims=True)
    padded = ((hist + 7) // 8) * 8
    k64 = lax.broadcasted_iota(jnp.int32, (E, E), 1)
    e64 = lax.broadcasted_iota(jnp.int32, (E, E), 0)
    offc = jnp.sum(jnp.where(k64 < e64, padded, 0), axis=1, keepdims=True)

    aoff = jnp.sum(jnp.where(ec == u_r, offc, 0), axis=0, keepdims=True)
    rank = jnp.zeros((1, S), jnp.int32)
    for kb in range(S // RB):
        u_cb = u_col_ref[pl.ds(kb * RB, RB), :]
        k_idx = kb * RB + lax.broadcasted_iota(jnp.int32, (RB, S), 0)
        j_idx = lax.broadcasted_iota(jnp.int32, (RB, S), 1)
        m = (u_cb == u_r) & (k_idx < j_idx)
        rank = rank + jnp.sum(m.astype(jnp.int32), axis=0, keepdims=True)
    sp_row = aoff + rank
    sp_ref[...] = sp_row

    for ib in range(P2 // RB):
        i_idx = ib * RB + lax.broadcasted_iota(jnp.int32, (RB, S), 0)
        j_idx = lax.broadcasted_iota(jnp.int32, (RB, S), 1)
        pick = sp_row == i_idx
        g_ref[pl.ds(ib * RB, RB), :] = jnp.sum(
            jnp.where(pick, j_idx, 0), axis=1, keepdims=True)


def _expert_kernel(u_ref, xs_ref, b_ref, wl_ref, wr_ref, out_ref):
    e = pl.program_id(0)
    u = u_ref[...]
    ec = lax.broadcasted_iota(jnp.int32, (E, 1), 0)
    hist = jnp.sum((ec == u).astype(jnp.int32), axis=1, keepdims=True)
    padded = ((hist + 7) // 8) * 8
    off = jnp.sum(jnp.where(ec < e, padded, 0))
    off = pl.multiple_of(off, 8)
    cnt = jnp.sum((u == e).astype(jnp.int32))
    nch = (cnt + (CHUNK - 1)) // CHUNK
    b = b_ref[0]
    H = D // 2

    def body(c, carry):
        s = off + c * CHUNK
        xa = xs_ref[pl.ds(s, CHUNK), :]
        accl = jnp.dot(xa, wl_ref[0], preferred_element_type=jnp.float32)
        accr = jnp.dot(xa, wr_ref[0], preferred_element_type=jnp.float32)
        out_ref[pl.ds(s, CHUNK), 0:H] = jnp.tanh(accl + b[:, 0:H])
        out_ref[pl.ds(s, CHUNK), H:D] = jnp.tanh(accr + b[:, H:D])
        return carry

    lax.fori_loop(0, nch, body, 0)


def _sc_row_gather(table, idx, n_out):
    n_idx = idx.shape[0]
    d = table.shape[1]
    mesh = plsc.VectorSubcoreMesh(core_axis_name="c", subcore_axis_name="s")
    nw = mesh.num_cores * mesh.num_subcores
    per = n_idx // nw
    half = per // 2

    @functools.partial(
        pl.kernel,
        out_type=jax.ShapeDtypeStruct((n_out, d), jnp.float32),
        mesh=mesh,
        scratch_types=[
            pltpu.VMEM((half,), jnp.int32),
            pltpu.VMEM((half,), jnp.int32),
            pltpu.VMEM((half, d), jnp.float32),
            pltpu.VMEM((half, d), jnp.float32),
            pltpu.SemaphoreType.DMA,
            pltpu.SemaphoreType.DMA,
        ],
    )
    def gk(table_hbm, idx_hbm, out_hbm, idx0, idx1, buf0, buf1, sem0, sem1):
        wid = lax.axis_index("s") * mesh.num_cores + lax.axis_index("c")
        base = wid * per
        pltpu.sync_copy(idx_hbm.at[pl.ds(base, half)], idx0)
        pltpu.sync_copy(idx_hbm.at[pl.ds(base + half, half)], idx1)
        c0 = pltpu.async_copy(table_hbm.at[idx0], buf0, sem0)
        c1 = pltpu.async_copy(table_hbm.at[idx1], buf1, sem1)
        c0.wait()
        pltpu.sync_copy(buf0, out_hbm.at[pl.ds(base, half)])
        c1.wait()
        pltpu.sync_copy(buf1, out_hbm.at[pl.ds(base + half, half)])

    return gk(table, idx)


def kernel(input, urls, trans, bias):
    x = input.reshape(S, D)
    u = urls.reshape(S).astype(jnp.int32)

    sp_row, g_col = pl.pallas_call(
        _rank_kernel,
        out_shape=(
            jax.ShapeDtypeStruct((1, S), jnp.int32),
            jax.ShapeDtypeStruct((P2, 1), jnp.int32),
        ),
    )(u.reshape(S, 1), u.reshape(1, S))
    sp = sp_row.reshape(S)
    g = g_col.reshape(P2)

    xs = _sc_row_gather(x, g, XROWS)

    out_sorted = pl.pallas_call(
        _expert_kernel,
        grid=(E,),
        in_specs=[
            pl.BlockSpec((1, S), lambda e: (0, 0)),
            pl.BlockSpec((XROWS, D), lambda e: (0, 0)),
            pl.BlockSpec((1, 1, D), lambda e: (e, 0, 0)),
            pl.BlockSpec((1, D, D // 2), lambda e: (e, 0, 0)),
            pl.BlockSpec((1, D, D // 2), lambda e: (e, 0, 1)),
        ],
        out_specs=pl.BlockSpec((XROWS, D), lambda e: (0, 0)),
        out_shape=jax.ShapeDtypeStruct((XROWS, D), jnp.float32),
    )(u.reshape(1, S), xs, bias.reshape(E, 1, D), trans, trans)

    out = _sc_row_gather(out_sorted, sp, S)
    return out.reshape(input.shape)


# speedup vs baseline: 1.1746x; 1.1746x over previous
import functools

import jax
import jax.numpy as jnp
from jax import lax
from jax.experimental import pallas as pl
from jax.experimental.pallas import tpu as pltpu
from jax.experimental.pallas import tpu_sc as plsc

S = 2048
D = 1024
E = 64
CHUNK = 128
P2 = S + 8 * E
XROWS = P2 + CHUNK
RB = 256


def _rank_kernel(u_col_ref, u_row_ref, sp_ref, g_ref):
    u_r = u_row_ref[...]
    u_c = u_col_ref[...]
    er = lax.broadcasted_iota(jnp.int32, (1, E), 1)
    ec = lax.broadcasted_iota(jnp.int32, (E, 1), 0)

    hist = jnp.sum((u_c == er).astype(jnp.int32), axis=0, keepdims=True)
    padded = ((hist + 7) // 8) * 8
    k64 = lax.broadcasted_iota(jnp.int32, (E, E), 1)
    e64 = lax.broadcasted_iota(jnp.int32, (E, E), 0)
    offc = jnp.sum(jnp.where(k64 < e64, padded, 0), axis=1, keepdims=True)

    aoff = jnp.sum(jnp.where(ec == u_r, offc, 0), axis=0, keepdims=True)
    rank = jnp.zeros((1, S), jnp.int32)
    for kb in range(S // RB):
        u_cb = u_col_ref[pl.ds(kb * RB, RB), :]
        k_idx = kb * RB + lax.broadcasted_iota(jnp.int32, (RB, S), 0)
        j_idx = lax.broadcasted_iota(jnp.int32, (RB, S), 1)
        m = (u_cb == u_r) & (k_idx < j_idx)
        rank = rank + jnp.sum(m.astype(jnp.int32), axis=0, keepdims=True)
    sp_row = aoff + rank
    sp_ref[...] = sp_row

    for ib in range(P2 // RB):
        i_idx = ib * RB + lax.broadcasted_iota(jnp.int32, (RB, S), 0)
        j_idx = lax.broadcasted_iota(jnp.int32, (RB, S), 1)
        pick = sp_row == i_idx
        picked = jnp.sum(jnp.where(pick, j_idx, 0), axis=1, keepdims=True)
        anyp = jnp.sum(pick.astype(jnp.int32), axis=1, keepdims=True) > 0
        fill = (ib * RB
                + lax.broadcasted_iota(jnp.int32, (RB, 1), 0)) & (S - 1)
        g_ref[pl.ds(ib * RB, RB), :] = jnp.where(anyp, picked, fill)


def _expert_kernel(u_ref, xs_ref, b_ref, wl_ref, wr_ref, out_ref):
    e = pl.program_id(0)
    u = u_ref[...]
    ec = lax.broadcasted_iota(jnp.int32, (E, 1), 0)
    hist = jnp.sum((ec == u).astype(jnp.int32), axis=1, keepdims=True)
    padded = ((hist + 7) // 8) * 8
    off = jnp.sum(jnp.where(ec < e, padded, 0))
    off = pl.multiple_of(off, 8)
    cnt = jnp.sum((u == e).astype(jnp.int32))
    nch = (cnt + (CHUNK - 1)) // CHUNK
    b = b_ref[0]
    H = D // 2

    def body(c, carry):
        s = off + c * CHUNK
        xa = xs_ref[pl.ds(s, CHUNK), :]
        accl = jnp.dot(xa, wl_ref[0], preferred_element_type=jnp.float32)
        accr = jnp.dot(xa, wr_ref[0], preferred_element_type=jnp.float32)
        out_ref[pl.ds(s, CHUNK), 0:H] = jnp.tanh(accl + b[:, 0:H])
        out_ref[pl.ds(s, CHUNK), H:D] = jnp.tanh(accr + b[:, H:D])
        return carry

    lax.fori_loop(0, nch, body, 0)


def _sc_row_gather(table, idx, n_out):
    n_idx = idx.shape[0]
    d = table.shape[1]
    mesh = plsc.VectorSubcoreMesh(core_axis_name="c", subcore_axis_name="s")
    nw = mesh.num_cores * mesh.num_subcores
    per = n_idx // nw
    half = per // 2

    @functools.partial(
        pl.kernel,
        out_type=jax.ShapeDtypeStruct((n_out, d), jnp.float32),
        mesh=mesh,
        scratch_types=[
            pltpu.VMEM((half,), jnp.int32),
            pltpu.VMEM((half,), jnp.int32),
            pltpu.VMEM((half, d), jnp.float32),
            pltpu.VMEM((half, d), jnp.float32),
            pltpu.SemaphoreType.DMA,
            pltpu.SemaphoreType.DMA,
        ],
    )
    def gk(table_hbm, idx_hbm, out_hbm, idx0, idx1, buf0, buf1, sem0, sem1):
        wid = lax.axis_index("s") * mesh.num_cores + lax.axis_index("c")
        base = wid * per
        pltpu.sync_copy(idx_hbm.at[pl.ds(base, half)], idx0)
        pltpu.sync_copy(idx_hbm.at[pl.ds(base + half, half)], idx1)
        c0 = pltpu.async_copy(table_hbm.at[idx0], buf0, sem0)
        c1 = pltpu.async_copy(table_hbm.at[idx1], buf1, sem1)
        c0.wait()
        pltpu.sync_copy(buf0, out_hbm.at[pl.ds(base, half)])
        c1.wait()
        pltpu.sync_copy(buf1, out_hbm.at[pl.ds(base + half, half)])

    return gk(table, idx)


def kernel(input, urls, trans, bias):
    x = input.reshape(S, D)
    u = urls.reshape(S).astype(jnp.int32)

    sp_row, g_col = pl.pallas_call(
        _rank_kernel,
        out_shape=(
            jax.ShapeDtypeStruct((1, S), jnp.int32),
            jax.ShapeDtypeStruct((P2, 1), jnp.int32),
        ),
    )(u.reshape(S, 1), u.reshape(1, S))
    sp = sp_row.reshape(S)
    g = g_col.reshape(P2)

    xs = _sc_row_gather(x, g, XROWS)

    out_sorted = pl.pallas_call(
        _expert_kernel,
        grid=(E,),
        in_specs=[
            pl.BlockSpec((1, S), lambda e: (0, 0)),
            pl.BlockSpec((XROWS, D), lambda e: (0, 0)),
            pl.BlockSpec((1, 1, D), lambda e: (e, 0, 0)),
            pl.BlockSpec((1, D, D // 2), lambda e: (e, 0, 0)),
            pl.BlockSpec((1, D, D // 2), lambda e: (e, 0, 1)),
        ],
        out_specs=pl.BlockSpec((XROWS, D), lambda e: (0, 0)),
        out_shape=jax.ShapeDtypeStruct((XROWS, D), jnp.float32),
    )(u.reshape(1, S), xs, bias.reshape(E, 1, D), trans, trans)

    out = _sc_row_gather(out_sorted, sp, S)
    return out.reshape(input.shape)


# speedup vs baseline: 1.3030x; 1.1094x over previous
import functools

import jax
import jax.numpy as jnp
from jax import lax
from jax.experimental import pallas as pl
from jax.experimental.pallas import tpu as pltpu
from jax.experimental.pallas import tpu_sc as plsc

S = 2048
D = 1024
E = 64
CHUNK = 128
P2 = S + 8 * E
XROWS = P2 + CHUNK
RB = 256


def _rank_kernel(u_col_ref, u_row_ref, sp_ref, g_ref):
    u_r = u_row_ref[...]
    u_c = u_col_ref[...]
    er = lax.broadcasted_iota(jnp.int32, (1, E), 1)
    ec = lax.broadcasted_iota(jnp.int32, (E, 1), 0)

    hist = jnp.sum((u_c == er).astype(jnp.int32), axis=0, keepdims=True)
    padded = ((hist + 7) // 8) * 8
    k64 = lax.broadcasted_iota(jnp.int32, (E, E), 1)
    e64 = lax.broadcasted_iota(jnp.int32, (E, E), 0)
    offc = jnp.sum(jnp.where(k64 < e64, padded, 0), axis=1, keepdims=True)

    aoff = jnp.sum(jnp.where(ec == u_r, offc, 0), axis=0, keepdims=True)
    rank = jnp.zeros((1, S), jnp.int32)
    for kb in range(S // RB):
        u_cb = u_col_ref[pl.ds(kb * RB, RB), :]
        k_idx = kb * RB + lax.broadcasted_iota(jnp.int32, (RB, S), 0)
        j_idx = lax.broadcasted_iota(jnp.int32, (RB, S), 1)
        m = (u_cb == u_r) & (k_idx < j_idx)
        rank = rank + jnp.sum(m.astype(jnp.int32), axis=0, keepdims=True)
    sp_row = aoff + rank
    sp_ref[...] = sp_row

    for ib in range(P2 // RB):
        i_idx = ib * RB + lax.broadcasted_iota(jnp.int32, (RB, S), 0)
        j_idx = lax.broadcasted_iota(jnp.int32, (RB, S), 1)
        pick = sp_row == i_idx
        picked = jnp.sum(jnp.where(pick, j_idx, 0), axis=1, keepdims=True)
        anyp = jnp.sum(pick.astype(jnp.int32), axis=1, keepdims=True) > 0
        fill = (ib * RB
                + lax.broadcasted_iota(jnp.int32, (RB, 1), 0)) & (S - 1)
        g_ref[pl.ds(ib * RB, RB), :] = jnp.where(anyp, picked, fill)


EPG = 2


def _expert_kernel(u_ref, xs_ref, b_ref, w_ref, out_ref):
    i = pl.program_id(0)
    u = u_ref[...]
    ec = lax.broadcasted_iota(jnp.int32, (E, 1), 0)
    hist = jnp.sum((ec == u).astype(jnp.int32), axis=1, keepdims=True)
    padded = ((hist + 7) // 8) * 8

    for t in range(EPG):
        e = i * EPG + t
        off = jnp.sum(jnp.where(ec < e, padded, 0))
        off = pl.multiple_of(off, 8)
        cnt = jnp.sum((u == e).astype(jnp.int32))
        nch = (cnt + (CHUNK - 1)) // CHUNK
        b = b_ref[t]

        def body(c, carry):
            s = off + c * CHUNK
            xa = xs_ref[pl.ds(s, CHUNK), :]
            acc = jnp.dot(xa, w_ref[t], preferred_element_type=jnp.float32)
            out_ref[pl.ds(s, CHUNK), :] = jnp.tanh(acc + b)
            return carry

        lax.fori_loop(0, nch, body, 0)


def _sc_row_gather(table, idx, n_out):
    n_idx = idx.shape[0]
    d = table.shape[1]
    mesh = plsc.VectorSubcoreMesh(core_axis_name="c", subcore_axis_name="s")
    nw = mesh.num_cores * mesh.num_subcores
    per = n_idx // nw
    half = per // 2

    @functools.partial(
        pl.kernel,
        out_type=jax.ShapeDtypeStruct((n_out, d), jnp.float32),
        mesh=mesh,
        scratch_types=[
            pltpu.VMEM((half,), jnp.int32),
            pltpu.VMEM((half,), jnp.int32),
            pltpu.VMEM((half, d), jnp.float32),
            pltpu.VMEM((half, d), jnp.float32),
            pltpu.SemaphoreType.DMA,
            pltpu.SemaphoreType.DMA,
        ],
    )
    def gk(table_hbm, idx_hbm, out_hbm, idx0, idx1, buf0, buf1, sem0, sem1):
        wid = lax.axis_index("s") * mesh.num_cores + lax.axis_index("c")
        base = wid * per
        pltpu.sync_copy(idx_hbm.at[pl.ds(base, half)], idx0)
        pltpu.sync_copy(idx_hbm.at[pl.ds(base + half, half)], idx1)
        c0 = pltpu.async_copy(table_hbm.at[idx0], buf0, sem0)
        c1 = pltpu.async_copy(table_hbm.at[idx1], buf1, sem1)
        c0.wait()
        pltpu.sync_copy(buf0, out_hbm.at[pl.ds(base, half)])
        c1.wait()
        pltpu.sync_copy(buf1, out_hbm.at[pl.ds(base + half, half)])

    return gk(table, idx)


def kernel(input, urls, trans, bias):
    x = input.reshape(S, D)
    u = urls.reshape(S).astype(jnp.int32)

    sp_row, g_col = pl.pallas_call(
        _rank_kernel,
        out_shape=(
            jax.ShapeDtypeStruct((1, S), jnp.int32),
            jax.ShapeDtypeStruct((P2, 1), jnp.int32),
        ),
    )(u.reshape(S, 1), u.reshape(1, S))
    sp = sp_row.reshape(S)
    g = g_col.reshape(P2)

    xs = _sc_row_gather(x, g, XROWS)

    out_sorted = pl.pallas_call(
        _expert_kernel,
        grid=(E // EPG,),
        in_specs=[
            pl.BlockSpec((1, S), lambda i: (0, 0)),
            pl.BlockSpec((XROWS, D), lambda i: (0, 0)),
            pl.BlockSpec((EPG, 1, D), lambda i: (i, 0, 0)),
            pl.BlockSpec((EPG, D, D), lambda i: (i, 0, 0)),
        ],
        out_specs=pl.BlockSpec((XROWS, D), lambda i: (0, 0)),
        out_shape=jax.ShapeDtypeStruct((XROWS, D), jnp.float32),
    )(u.reshape(1, S), xs, bias.reshape(E, 1, D), trans)

    out = _sc_row_gather(out_sorted, sp, S)
    return out.reshape(input.shape)


# speedup vs baseline: 1.3036x; 1.0005x over previous
import functools

import jax
import jax.numpy as jnp
from jax import lax
from jax.experimental import pallas as pl
from jax.experimental.pallas import tpu as pltpu
from jax.experimental.pallas import tpu_sc as plsc

S = 2048
D = 1024
E = 64
CHUNK = 128
P2 = S + 8 * E
XROWS = P2 + CHUNK
RB = 256


def _rank_kernel(u_col_ref, u_row_ref, sp_ref, g_ref):
    u_r = u_row_ref[...]
    u_c = u_col_ref[...]
    er = lax.broadcasted_iota(jnp.int32, (1, E), 1)
    ec = lax.broadcasted_iota(jnp.int32, (E, 1), 0)

    hist = jnp.sum((u_c == er).astype(jnp.int32), axis=0, keepdims=True)
    padded = ((hist + 7) // 8) * 8
    k64 = lax.broadcasted_iota(jnp.int32, (E, E), 1)
    e64 = lax.broadcasted_iota(jnp.int32, (E, E), 0)
    offc = jnp.sum(jnp.where(k64 < e64, padded, 0), axis=1, keepdims=True)

    aoff = jnp.sum(jnp.where(ec == u_r, offc, 0), axis=0, keepdims=True)
    rank = jnp.zeros((1, S), jnp.int32)
    for kb in range(S // RB):
        u_cb = u_col_ref[pl.ds(kb * RB, RB), :]
        k_idx = kb * RB + lax.broadcasted_iota(jnp.int32, (RB, S), 0)
        j_idx = lax.broadcasted_iota(jnp.int32, (RB, S), 1)
        m = (u_cb == u_r) & (k_idx < j_idx)
        rank = rank + jnp.sum(m.astype(jnp.int32), axis=0, keepdims=True)
    sp_row = aoff + rank
    sp_ref[...] = sp_row

    for ib in range(P2 // RB):
        i_idx = ib * RB + lax.broadcasted_iota(jnp.int32, (RB, S), 0)
        j_idx = lax.broadcasted_iota(jnp.int32, (RB, S), 1)
        pick = sp_row == i_idx
        picked = jnp.sum(jnp.where(pick, j_idx, 0), axis=1, keepdims=True)
        anyp = jnp.sum(pick.astype(jnp.int32), axis=1, keepdims=True) > 0
        fill = (ib * RB
                + lax.broadcasted_iota(jnp.int32, (RB, 1), 0)) & (S - 1)
        g_ref[pl.ds(ib * RB, RB), :] = jnp.where(anyp, picked, fill)


EPG = 4


def _expert_kernel(u_ref, xs_ref, b_ref, w_ref, out_ref):
    i = pl.program_id(0)
    u = u_ref[...]
    ec = lax.broadcasted_iota(jnp.int32, (E, 1), 0)
    hist = jnp.sum((ec == u).astype(jnp.int32), axis=1, keepdims=True)
    padded = ((hist + 7) // 8) * 8

    for t in range(EPG):
        e = i * EPG + t
        off = jnp.sum(jnp.where(ec < e, padded, 0))
        off = pl.multiple_of(off, 8)
        cnt = jnp.sum((u == e).astype(jnp.int32))
        nch = (cnt + (CHUNK - 1)) // CHUNK
        b = b_ref[t]

        def body(c, carry):
            s = off + c * CHUNK
            xa = xs_ref[pl.ds(s, CHUNK), :]
            acc = jnp.dot(xa, w_ref[t], preferred_element_type=jnp.float32)
            out_ref[pl.ds(s, CHUNK), :] = jnp.tanh(acc + b)
            return carry

        lax.fori_loop(0, nch, body, 0)


def _sc_row_gather(table, idx, n_out):
    n_idx = idx.shape[0]
    d = table.shape[1]
    mesh = plsc.VectorSubcoreMesh(core_axis_name="c", subcore_axis_name="s")
    nw = mesh.num_cores * mesh.num_subcores
    per = n_idx // nw
    half = per // 2

    @functools.partial(
        pl.kernel,
        out_type=jax.ShapeDtypeStruct((n_out, d), jnp.float32),
        mesh=mesh,
        scratch_types=[
            pltpu.VMEM((half,), jnp.int32),
            pltpu.VMEM((half,), jnp.int32),
            pltpu.VMEM((half, d), jnp.float32),
            pltpu.VMEM((half, d), jnp.float32),
            pltpu.SemaphoreType.DMA,
            pltpu.SemaphoreType.DMA,
        ],
    )
    def gk(table_hbm, idx_hbm, out_hbm, idx0, idx1, buf0, buf1, sem0, sem1):
        wid = lax.axis_index("s") * mesh.num_cores + lax.axis_index("c")
        base = wid * per
        pltpu.sync_copy(idx_hbm.at[pl.ds(base, half)], idx0)
        pltpu.sync_copy(idx_hbm.at[pl.ds(base + half, half)], idx1)
        c0 = pltpu.async_copy(table_hbm.at[idx0], buf0, sem0)
        c1 = pltpu.async_copy(table_hbm.at[idx1], buf1, sem1)
        c0.wait()
        pltpu.sync_copy(buf0, out_hbm.at[pl.ds(base, half)])
        c1.wait()
        pltpu.sync_copy(buf1, out_hbm.at[pl.ds(base + half, half)])

    return gk(table, idx)


def kernel(input, urls, trans, bias):
    x = input.reshape(S, D)
    u = urls.reshape(S).astype(jnp.int32)

    sp_row, g_col = pl.pallas_call(
        _rank_kernel,
        out_shape=(
            jax.ShapeDtypeStruct((1, S), jnp.int32),
            jax.ShapeDtypeStruct((P2, 1), jnp.int32),
        ),
    )(u.reshape(S, 1), u.reshape(1, S))
    sp = sp_row.reshape(S)
    g = g_col.reshape(P2)

    xs = _sc_row_gather(x, g, XROWS)

    out_sorted = pl.pallas_call(
        _expert_kernel,
        grid=(E // EPG,),
        in_specs=[
            pl.BlockSpec((1, S), lambda i: (0, 0)),
            pl.BlockSpec((XROWS, D), lambda i: (0, 0)),
            pl.BlockSpec((EPG, 1, D), lambda i: (i, 0, 0)),
            pl.BlockSpec((EPG, D, D), lambda i: (i, 0, 0)),
        ],
        out_specs=pl.BlockSpec((XROWS, D), lambda i: (0, 0)),
        out_shape=jax.ShapeDtypeStruct((XROWS, D), jnp.float32),
    )(u.reshape(1, S), xs, bias.reshape(E, 1, D), trans)

    out = _sc_row_gather(out_sorted, sp, S)
    return out.reshape(input.shape)


# speedup vs baseline: 1.4016x; 1.0752x over previous
import functools

import jax
import jax.numpy as jnp
from jax import lax
from jax.experimental import pallas as pl
from jax.experimental.pallas import tpu as pltpu
from jax.experimental.pallas import tpu_sc as plsc

S = 2048
D = 1024
E = 64
CHUNK = 128
P2 = S + 8 * E
XROWS = P2 + CHUNK
RB = 256
EPG = 2


def _rank_kernel(u_col_ref, u_row_ref, sp_ref):
    u_r = u_row_ref[...]
    u_c = u_col_ref[...]
    er = lax.broadcasted_iota(jnp.int32, (1, E), 1)
    ec = lax.broadcasted_iota(jnp.int32, (E, 1), 0)

    hist = jnp.sum((u_c == er).astype(jnp.int32), axis=0, keepdims=True)
    padded = ((hist + 7) // 8) * 8
    k64 = lax.broadcasted_iota(jnp.int32, (E, E), 1)
    e64 = lax.broadcasted_iota(jnp.int32, (E, E), 0)
    offc = jnp.sum(jnp.where(k64 < e64, padded, 0), axis=1, keepdims=True)

    aoff = jnp.sum(jnp.where(ec == u_r, offc, 0), axis=0, keepdims=True)
    rank = jnp.zeros((1, S), jnp.int32)
    for kb in range(S // RB):
        u_cb = u_col_ref[pl.ds(kb * RB, RB), :]
        k_idx = kb * RB + lax.broadcasted_iota(jnp.int32, (RB, S), 0)
        j_idx = lax.broadcasted_iota(jnp.int32, (RB, S), 1)
        m = (u_cb == u_r) & (k_idx < j_idx)
        rank = rank + jnp.sum(m.astype(jnp.int32), axis=0, keepdims=True)
    sp_ref[...] = aoff + rank


def _expert_kernel(u_ref, xs_ref, b_ref, w_ref, out_ref):
    i = pl.program_id(0)
    u = u_ref[...]
    b_full = b_ref[...]
    ec = lax.broadcasted_iota(jnp.int32, (E, 1), 0)
    hist = jnp.sum((ec == u).astype(jnp.int32), axis=1, keepdims=True)
    padded = ((hist + 7) // 8) * 8

    for t in range(EPG):
        e = i * EPG + t
        off = jnp.sum(jnp.where(ec < e, padded, 0))
        off = pl.multiple_of(off, 8)
        cnt = jnp.sum((u == e).astype(jnp.int32))
        nch = (cnt + (CHUNK - 1)) // CHUNK
        b = jnp.sum(jnp.where(ec == e, b_full, 0.0), axis=0, keepdims=True)

        def body(c, carry):
            s = off + c * CHUNK
            xa = xs_ref[pl.ds(s, CHUNK), :]
            acc = jnp.dot(xa, w_ref[t], preferred_element_type=jnp.float32)
            out_ref[pl.ds(s, CHUNK), :] = jnp.tanh(acc + b)
            return carry

        lax.fori_loop(0, nch, body, 0)


def _sc_scatter_rows(x, sp, n_out):
    n, d = x.shape
    mesh = plsc.VectorSubcoreMesh(core_axis_name="c", subcore_axis_name="s")
    nw = mesh.num_cores * mesh.num_subcores
    per = n // nw

    @functools.partial(
        pl.kernel,
        out_type=jax.ShapeDtypeStruct((n_out, d), jnp.float32),
        mesh=mesh,
        scratch_types=[
            pltpu.VMEM((per,), jnp.int32),
            pltpu.VMEM((per, d), jnp.float32),
            pltpu.SemaphoreType.DMA,
        ],
    )
    def sk(x_hbm, sp_hbm, out_hbm, idx_v, rows_v, sem):
        wid = lax.axis_index("s") * mesh.num_cores + lax.axis_index("c")
        base = wid * per
        pltpu.sync_copy(sp_hbm.at[pl.ds(base, per)], idx_v)
        pltpu.sync_copy(x_hbm.at[pl.ds(base, per)], rows_v)
        pltpu.async_copy(rows_v, out_hbm.at[idx_v], sem).wait()

    return sk(x, sp)


def _sc_row_gather(table, idx, n_out):
    n_idx = idx.shape[0]
    d = table.shape[1]
    mesh = plsc.VectorSubcoreMesh(core_axis_name="c", subcore_axis_name="s")
    nw = mesh.num_cores * mesh.num_subcores
    per = n_idx // nw

    @functools.partial(
        pl.kernel,
        out_type=jax.ShapeDtypeStruct((n_out, d), jnp.float32),
        mesh=mesh,
        scratch_types=[
            pltpu.VMEM((per,), jnp.int32),
            pltpu.VMEM((per, d), jnp.float32),
            pltpu.SemaphoreType.DMA,
        ],
    )
    def gk(table_hbm, idx_hbm, out_hbm, idx_v, rows_v, sem):
        wid = lax.axis_index("s") * mesh.num_cores + lax.axis_index("c")
        base = wid * per
        pltpu.sync_copy(idx_hbm.at[pl.ds(base, per)], idx_v)
        pltpu.async_copy(table_hbm.at[idx_v], rows_v, sem).wait()
        pltpu.sync_copy(rows_v, out_hbm.at[pl.ds(base, per)])

    return gk(table, idx)


def kernel(input, urls, trans, bias):
    x = input.reshape(S, D)
    u = urls.reshape(S).astype(jnp.int32)

    sp_row = pl.pallas_call(
        _rank_kernel,
        out_shape=jax.ShapeDtypeStruct((1, S), jnp.int32),
    )(u.reshape(S, 1), u.reshape(1, S))
    sp = sp_row.reshape(S)

    xs = _sc_scatter_rows(x, sp, XROWS)

    out_sorted = pl.pallas_call(
        _expert_kernel,
        grid=(E // EPG,),
        in_specs=[
            pl.BlockSpec((1, S), lambda i: (0, 0)),
            pl.BlockSpec((XROWS, D), lambda i: (0, 0)),
            pl.BlockSpec((E, D), lambda i: (0, 0)),
            pl.BlockSpec((EPG, D, D), lambda i: (i, 0, 0)),
        ],
        out_specs=pl.BlockSpec((XROWS, D), lambda i: (0, 0)),
        out_shape=jax.ShapeDtypeStruct((XROWS, D), jnp.float32),
    )(u.reshape(1, S), xs, bias, trans)

    out = _sc_row_gather(out_sorted, sp, S)
    return out.reshape(input.shape)
